# baseline (speedup 1.0000x reference)
import jax
import jax.numpy as jnp
from jax.experimental import pallas as pl
from jax.experimental.pallas import tpu as pltpu

BATCH = 8
WIDTH = 32
N_NODE = 2048
N_REL = 4
JB = 1024
NJ = N_NODE // JB


def _rowscale(col):
    return jnp.concatenate([col] * BATCH, axis=0)


def _lerp_kernel(db_ref, x_ref, w_ref, eq_ref, out_ref, xs1_ref, xs2_ref):
    r = pl.program_id(0)
    j = pl.program_id(1)
    step = r * NJ + j
    nsteps = N_REL * NJ

    @pl.when(j == 0)
    def _prep():
        wsm = jax.nn.softmax(w_ref[...], axis=1)

        def sel(base):
            c = wsm[:, base + N_REL - 1 : base + N_REL]
            for k in range(N_REL - 2, -1, -1):
                c = jnp.where(r == k, wsm[:, base + k : base + k + 1], c)
            return c

        x = x_ref[...]
        xs1_ref[...] = (x * _rowscale(sel(0))).astype(jnp.bfloat16)
        xs2_ref[...] = (x * _rowscale(sel(N_REL))).astype(jnp.bfloat16)

    d = db_ref[0].astype(jnp.bfloat16)

    y1 = jax.lax.dot_general(
        xs1_ref[:, pl.ds(j * JB, JB)], d,
        (((1,), (0,)), ((), ())), preferred_element_type=jnp.float32,
    )

    @pl.when(step == 0)
    def _first():
        out_ref[...] = y1

    @pl.when(step > 0)
    def _rest():
        out_ref[...] += y1

    y2 = jax.lax.dot_general(
        xs2_ref[...], d,
        (((1,), (1,)), ((), ())), preferred_element_type=jnp.float32,
    )
    out_ref[:, pl.ds(j * JB, JB)] += y2

    @pl.when(step == nsteps - 1)
    def _fin():
        eqsm = jax.nn.softmax(eq_ref[...], axis=1)
        eq0 = _rowscale(eqsm[:, 0:1])
        eq1 = _rowscale(eqsm[:, 1:2])
        acc = out_ref[...]
        out_ref[...] = (1.0 - jnp.exp(-acc)) * eq0 + x_ref[...] * eq1


@jax.jit
def kernel(inputs, database, weights, equity_weight):
    m = BATCH * WIDTH
    x = inputs.reshape(m, N_NODE)
    out2d = pl.pallas_call(
        _lerp_kernel,
        grid=(N_REL, NJ),
        in_specs=[
            pl.BlockSpec((1, JB, N_NODE), lambda r, j: (r, j, 0)),
            pl.BlockSpec((m, N_NODE), lambda r, j: (0, 0)),
            pl.BlockSpec((WIDTH, 2 * N_REL), lambda r, j: (0, 0)),
            pl.BlockSpec((WIDTH, 2), lambda r, j: (0, 0)),
        ],
        out_specs=pl.BlockSpec((m, N_NODE), lambda r, j: (0, 0)),
        out_shape=jax.ShapeDtypeStruct((m, N_NODE), jnp.float32),
        scratch_shapes=[
            pltpu.VMEM((m, N_NODE), jnp.bfloat16),
            pltpu.VMEM((m, N_NODE), jnp.bfloat16),
        ],
    )(database, x, weights, equity_weight)
    return out2d.reshape(BATCH, WIDTH, N_NODE)
